# baseline (speedup 1.0000x reference)
import functools

import jax
import jax.numpy as jnp
from jax import lax
from jax.experimental import pallas as pl
from jax.experimental.pallas import tpu as pltpu
from jax.experimental.pallas import tpu_sc as plsc

D = 64
C = 128
NB = 8
KA = 5


def _sc_gather(words3, word_table):
    nw, chunks, c = words3.shape
    assert c == C and chunks % NB == 0 and chunks >= 2 * NB
    n = nw * chunks * C
    per_w = chunks * C
    rounds = chunks // NB

    mesh = plsc.VectorSubcoreMesh(core_axis_name="c", subcore_axis_name="s")
    info = plsc.get_sparse_core_info()
    nc = info.num_cores
    assert nw == nc * info.num_subcores

    @functools.partial(
        pl.kernel,
        out_type=jax.ShapeDtypeStruct((n, D), jnp.float32),
        mesh=mesh,
        compiler_params=pltpu.CompilerParams(use_tc_tiling_on_sc=False),
        scratch_types=[
            pltpu.VMEM((chunks, C), jnp.int32),
            pltpu.VMEM((NB, C, D), jnp.float32),
        ]
        + [pltpu.SemaphoreType.DMA] * (2 * NB + 1),
    )
    def k(words_hbm, wtab_hbm, w_out, idxw, rw, *sems):
        gsem = sems[:NB]
        wsem = sems[NB:2 * NB]
        isem = sems[2 * NB]
        wid = lax.axis_index("s") * nc + lax.axis_index("c")
        base = wid * per_w

        pltpu.async_copy(words_hbm.at[wid], idxw, isem).wait()

        def start_gather(cl, b):
            pltpu.async_copy(wtab_hbm.at[idxw.at[cl]], rw.at[b], gsem[b])

        def drain_gather(b):
            pltpu.make_async_copy(wtab_hbm.at[pl.ds(0, C)], rw.at[b],
                                  gsem[b]).wait()

        def start_write(off, b):
            pltpu.async_copy(rw.at[b], w_out.at[pl.ds(off, C)], wsem[b])

        def drain_write(b):
            pltpu.make_async_copy(rw.at[b], w_out.at[pl.ds(0, C)],
                                  wsem[b]).wait()

        for j in range(KA):
            start_gather(j, j)

        def round_body(r, carry):
            for b in range(NB):
                i = r * NB + b
                bg = (b + KA) % NB

                @pl.when(i + KA < chunks)
                def _():
                    @pl.when(i >= NB - KA)
                    def _():
                        drain_write(bg)
                    start_gather(i + KA, bg)

                drain_gather(b)
                start_write(base + i * C, b)
            return carry

        lax.fori_loop(0, rounds, round_body, 0)
        for b in range(NB):
            drain_write(b)

    return k(words3, word_table)


def _epilogue_body(w_ref, words_ref, feats_ref, ftab_ref,
                   sum_ref, feat_ref, mask_ref):
    w2 = w_ref[...]
    wt = w2.T
    wordm = jnp.concatenate([wt[:D], wt[D:]], axis=1)
    f1 = feats_ref[0]
    bsz = f1.shape[1]
    onehot = (lax.broadcasted_iota(jnp.int32, (2 * D, bsz), 0) == f1).astype(
        jnp.float32)
    fe = jnp.dot(ftab_ref[...], onehot,
                 preferred_element_type=jnp.float32)
    sum_ref[...] = (wordm + fe)[None]
    feat_ref[...] = fe[None]
    mask_ref[...] = words_ref[...] != 0


def kernel(words, feats, word_table, feat_table):
    batch, seq = words.shape
    n = batch * seq
    nw = 32
    chunks = n // (nw * C)
    half = batch // 2

    words_t = words.T
    feats_t = feats.T
    wperm = (words_t.reshape(seq, 2, half).transpose(0, 2, 1)
             .reshape(seq, batch))
    w_rows = _sc_gather(wperm.reshape(nw, chunks, C), word_table)

    ftab_t = jnp.pad(feat_table, ((0, 2 * D - feat_table.shape[0]), (0, 0))).T
    st, ft, mask_t = pl.pallas_call(
        _epilogue_body,
        grid=(seq,),
        in_specs=[
            pl.BlockSpec((half, 2 * D), lambda i: (i, 0)),
            pl.BlockSpec((1, 1, batch), lambda i: (i, 0, 0)),
            pl.BlockSpec((1, 1, batch), lambda i: (i, 0, 0)),
            pl.BlockSpec((D, 2 * D), lambda i: (0, 0)),
        ],
        out_specs=[
            pl.BlockSpec((1, D, batch), lambda i: (i, 0, 0)),
            pl.BlockSpec((1, D, batch), lambda i: (i, 0, 0)),
            pl.BlockSpec((1, 1, batch), lambda i: (i, 0, 0)),
        ],
        out_shape=[
            jax.ShapeDtypeStruct((seq, D, batch), jnp.float32),
            jax.ShapeDtypeStruct((seq, D, batch), jnp.float32),
            jax.ShapeDtypeStruct((seq, 1, batch), jnp.bool_),
        ],
    )(w_rows.reshape(n // 2, 2 * D), words_t.reshape(seq, 1, batch),
      feats_t.reshape(seq, 1, batch), ftab_t)
    return (st.transpose(2, 0, 1), ft.transpose(2, 0, 1),
            mask_t.reshape(seq, batch).T, seq)


# speedup vs baseline: 1.0343x; 1.0343x over previous
import functools

import jax
import jax.numpy as jnp
from jax import lax
from jax.experimental import pallas as pl
from jax.experimental.pallas import tpu as pltpu
from jax.experimental.pallas import tpu_sc as plsc

D = 64
C = 128
NB = 4
KA = 2


def _sc_gather(words3, word_table):
    nw, chunks, c = words3.shape
    assert c == C and chunks % NB == 0 and chunks >= 2 * NB
    n = nw * chunks * C
    per_w = chunks * C
    rounds = chunks // NB

    mesh = plsc.VectorSubcoreMesh(core_axis_name="c", subcore_axis_name="s")
    info = plsc.get_sparse_core_info()
    nc = info.num_cores
    assert nw == nc * info.num_subcores

    @functools.partial(
        pl.kernel,
        out_type=jax.ShapeDtypeStruct((n, 2 * D), jnp.float32),
        mesh=mesh,
        compiler_params=pltpu.CompilerParams(use_tc_tiling_on_sc=False),
        scratch_types=[
            pltpu.VMEM((chunks, C), jnp.int32),
            pltpu.VMEM((NB, C, 2 * D), jnp.float32),
        ]
        + [pltpu.SemaphoreType.DMA] * (2 * NB + 1),
    )
    def k(words_hbm, wtab_hbm, w_out, idxw, rw, *sems):
        gsem = sems[:NB]
        wsem = sems[NB:2 * NB]
        isem = sems[2 * NB]
        wid = lax.axis_index("s") * nc + lax.axis_index("c")
        base = wid * per_w

        pltpu.async_copy(words_hbm.at[wid], idxw, isem).wait()

        def start_gather(cl, b):
            pltpu.async_copy(wtab_hbm.at[idxw.at[cl]], rw.at[b], gsem[b])

        def drain_gather(b):
            pltpu.make_async_copy(wtab_hbm.at[pl.ds(0, C)], rw.at[b],
                                  gsem[b]).wait()

        def start_write(off, b):
            pltpu.async_copy(rw.at[b], w_out.at[pl.ds(off, C)], wsem[b])

        def drain_write(b):
            pltpu.make_async_copy(rw.at[b], w_out.at[pl.ds(0, C)],
                                  wsem[b]).wait()

        for j in range(KA):
            start_gather(j, j)

        def round_body(r, carry):
            for b in range(NB):
                i = r * NB + b
                bg = (b + KA) % NB

                @pl.when(i + KA < chunks)
                def _():
                    @pl.when(i >= NB - KA)
                    def _():
                        drain_write(bg)
                    start_gather(i + KA, bg)

                drain_gather(b)
                start_write(base + i * C, b)
            return carry

        lax.fori_loop(0, rounds, round_body, 0)
        for b in range(NB):
            drain_write(b)

    return k(words3, word_table)


def _epilogue_body(w_ref, words_ref, feats_ref, ftab_ref,
                   sum_ref, feat_ref, mask_ref):
    wordm = w_ref[...].T[:D]
    f1 = feats_ref[0]
    bsz = f1.shape[1]
    onehot = (lax.broadcasted_iota(jnp.int32, (2 * D, bsz), 0) == f1).astype(
        jnp.float32)
    fe = jnp.dot(ftab_ref[...], onehot,
                 preferred_element_type=jnp.float32)
    sum_ref[...] = (wordm + fe)[None]
    feat_ref[...] = fe[None]
    mask_ref[...] = words_ref[...] != 0


def kernel(words, feats, word_table, feat_table):
    batch, seq = words.shape
    n = batch * seq
    nw = 32
    chunks = n // (nw * C)
    words_t = words.T
    feats_t = feats.T
    wtab128 = jnp.pad(word_table, ((0, 0), (0, D)))
    w_rows = _sc_gather(words_t.reshape(nw, chunks, C), wtab128)

    ftab_t = jnp.pad(feat_table, ((0, 2 * D - feat_table.shape[0]), (0, 0))).T
    st, ft, mask_t = pl.pallas_call(
        _epilogue_body,
        grid=(seq,),
        in_specs=[
            pl.BlockSpec((batch, 2 * D), lambda i: (i, 0)),
            pl.BlockSpec((1, 1, batch), lambda i: (i, 0, 0)),
            pl.BlockSpec((1, 1, batch), lambda i: (i, 0, 0)),
            pl.BlockSpec((D, 2 * D), lambda i: (0, 0)),
        ],
        out_specs=[
            pl.BlockSpec((1, D, batch), lambda i: (i, 0, 0)),
            pl.BlockSpec((1, D, batch), lambda i: (i, 0, 0)),
            pl.BlockSpec((1, 1, batch), lambda i: (i, 0, 0)),
        ],
        out_shape=[
            jax.ShapeDtypeStruct((seq, D, batch), jnp.float32),
            jax.ShapeDtypeStruct((seq, D, batch), jnp.float32),
            jax.ShapeDtypeStruct((seq, 1, batch), jnp.bool_),
        ],
    )(w_rows, words_t.reshape(seq, 1, batch),
      feats_t.reshape(seq, 1, batch), ftab_t)
    return (st.transpose(2, 0, 1), ft.transpose(2, 0, 1),
            mask_t.reshape(seq, batch).T, seq)


# speedup vs baseline: 1.0344x; 1.0001x over previous
import functools

import jax
import jax.numpy as jnp
from jax import lax
from jax.experimental import pallas as pl
from jax.experimental.pallas import tpu as pltpu
from jax.experimental.pallas import tpu_sc as plsc

D = 64
C = 128
NB = 5
KA = 3


def _sc_gather(words3, word_table):
    nw, chunks, c = words3.shape
    assert c == C and chunks % NB == 0 and chunks >= 2 * NB
    n = nw * chunks * C
    per_w = chunks * C
    rounds = chunks // NB

    mesh = plsc.VectorSubcoreMesh(core_axis_name="c", subcore_axis_name="s")
    info = plsc.get_sparse_core_info()
    nc = info.num_cores
    assert nw == nc * info.num_subcores

    @functools.partial(
        pl.kernel,
        out_type=jax.ShapeDtypeStruct((n, 2 * D), jnp.float32),
        mesh=mesh,
        compiler_params=pltpu.CompilerParams(use_tc_tiling_on_sc=False),
        scratch_types=[
            pltpu.VMEM((chunks, C), jnp.int32),
            pltpu.VMEM((NB, C, 2 * D), jnp.float32),
        ]
        + [pltpu.SemaphoreType.DMA] * (2 * NB + 1),
    )
    def k(words_hbm, wtab_hbm, w_out, idxw, rw, *sems):
        gsem = sems[:NB]
        wsem = sems[NB:2 * NB]
        isem = sems[2 * NB]
        wid = lax.axis_index("s") * nc + lax.axis_index("c")
        base = wid * per_w

        pltpu.async_copy(words_hbm.at[wid], idxw, isem).wait()

        def start_gather(cl, b):
            pltpu.async_copy(wtab_hbm.at[idxw.at[cl]], rw.at[b], gsem[b])

        def drain_gather(b):
            pltpu.make_async_copy(wtab_hbm.at[pl.ds(0, C)], rw.at[b],
                                  gsem[b]).wait()

        def start_write(off, b):
            pltpu.async_copy(rw.at[b], w_out.at[pl.ds(off, C)], wsem[b])

        def drain_write(b):
            pltpu.make_async_copy(rw.at[b], w_out.at[pl.ds(0, C)],
                                  wsem[b]).wait()

        for j in range(KA):
            start_gather(j, j)

        def round_body(r, carry):
            for b in range(NB):
                i = r * NB + b
                bg = (b + KA) % NB

                @pl.when(i + KA < chunks)
                def _():
                    @pl.when(i >= NB - KA)
                    def _():
                        drain_write(bg)
                    start_gather(i + KA, bg)

                drain_gather(b)
                start_write(base + i * C, b)
            return carry

        lax.fori_loop(0, rounds, round_body, 0)
        for b in range(NB):
            drain_write(b)

    return k(words3, word_table)


def _epilogue_body(w_ref, words_ref, feats_ref, ftab_ref,
                   sum_ref, feat_ref, mask_ref):
    wordm = w_ref[...].T[:D]
    f1 = feats_ref[0]
    bsz = f1.shape[1]
    onehot = (lax.broadcasted_iota(jnp.int32, (2 * D, bsz), 0) == f1).astype(
        jnp.float32)
    fe = jnp.dot(ftab_ref[...], onehot,
                 preferred_element_type=jnp.float32)
    sum_ref[...] = (wordm + fe)[None]
    feat_ref[...] = fe[None]
    mask_ref[...] = words_ref[...] != 0


def kernel(words, feats, word_table, feat_table):
    batch, seq = words.shape
    n = batch * seq
    nw = 32
    chunks = n // (nw * C)
    words_t = words.T
    feats_t = feats.T
    wtab128 = jnp.pad(word_table, ((0, 0), (0, D)))
    w_rows = _sc_gather(words_t.reshape(nw, chunks, C), wtab128)

    ftab_t = jnp.pad(feat_table, ((0, 2 * D - feat_table.shape[0]), (0, 0))).T
    st, ft, mask_t = pl.pallas_call(
        _epilogue_body,
        grid=(seq,),
        in_specs=[
            pl.BlockSpec((batch, 2 * D), lambda i: (i, 0)),
            pl.BlockSpec((1, 1, batch), lambda i: (i, 0, 0)),
            pl.BlockSpec((1, 1, batch), lambda i: (i, 0, 0)),
            pl.BlockSpec((D, 2 * D), lambda i: (0, 0)),
        ],
        out_specs=[
            pl.BlockSpec((1, D, batch), lambda i: (i, 0, 0)),
            pl.BlockSpec((1, D, batch), lambda i: (i, 0, 0)),
            pl.BlockSpec((1, 1, batch), lambda i: (i, 0, 0)),
        ],
        out_shape=[
            jax.ShapeDtypeStruct((seq, D, batch), jnp.float32),
            jax.ShapeDtypeStruct((seq, D, batch), jnp.float32),
            jax.ShapeDtypeStruct((seq, 1, batch), jnp.bool_),
        ],
    )(w_rows, words_t.reshape(seq, 1, batch),
      feats_t.reshape(seq, 1, batch), ftab_t)
    return (st.transpose(2, 0, 1), ft.transpose(2, 0, 1),
            mask_t.reshape(seq, batch).T, seq)
